# baseline (speedup 1.0000x reference)
import functools

import jax
import jax.numpy as jnp
from jax import lax
from jax.experimental import pallas as pl
from jax.experimental.pallas import tpu as pltpu
from jax.experimental.pallas import tpu_sc as plsc

N = 10000
E = 320000
D = 128

NC = 2
NS = 16
NW = NC * NS

CW = 80
EPT = E // NW
K = EPT // CW
NPAD = 10240
ZD = NPAD // NS
CWA = 128
EPTA = 10240
KA = EPTA // CWA
WT = 5
WK = KA // WT
NPA = 10112
ZR = NPA // NS
NSPL = 5
K5 = K // NSPL
ESL = E // NSPL
EPT5 = EPT // NSPL

_f32 = jnp.float32
_mesh = plsc.VectorSubcoreMesh(core_axis_name="c", subcore_axis_name="s")


def _deg_body(src2d, dst2d, zeros1d, out_hbm, idx_s, idx_d, ones_v,
              dout_sp, din_sp, sem, sem2):
    c = lax.axis_index("c")
    s = lax.axis_index("s")
    w = s * NC + c
    for k in range(CW // 16):
        ones_v[pl.ds(k * 16, 16)] = jnp.ones((16,), _f32)
    pltpu.sync_copy(zeros1d.at[pl.ds(s * ZD, ZD)], dout_sp.at[pl.ds(s * ZD, ZD)])
    pltpu.sync_copy(zeros1d.at[pl.ds(s * ZD, ZD)], din_sp.at[pl.ds(s * ZD, ZD)])
    pltpu.sync_copy(src2d.at[w], idx_s)
    pltpu.sync_copy(dst2d.at[w], idx_d)
    plsc.subcore_barrier()

    def body(j, carry):
        pltpu.async_copy(ones_v, dout_sp.at[idx_s.at[j]], sem, add=True)
        pltpu.async_copy(ones_v, din_sp.at[idx_d.at[j]], sem2, add=True)

        @pl.when(j >= 1)
        def _():
            pltpu.make_async_copy(ones_v, dout_sp.at[pl.ds(0, CW)], sem).wait()
            pltpu.make_async_copy(ones_v, din_sp.at[pl.ds(0, CW)], sem2).wait()
        return carry

    lax.fori_loop(0, K, body, 0)
    pltpu.make_async_copy(ones_v, dout_sp.at[pl.ds(0, CW)], sem).wait()
    pltpu.make_async_copy(ones_v, din_sp.at[pl.ds(0, CW)], sem2).wait()
    plsc.subcore_barrier()
    pltpu.sync_copy(dout_sp.at[pl.ds(s * ZD, ZD)], out_hbm.at[c, 0, pl.ds(s * ZD, ZD)])
    pltpu.sync_copy(din_sp.at[pl.ds(s * ZD, ZD)], out_hbm.at[c, 1, pl.ds(s * ZD, ZD)])


_deg_kernel = functools.partial(
    pl.kernel,
    out_type=jax.ShapeDtypeStruct((NC, 2, NPAD), _f32),
    mesh=_mesh,
    scratch_types=[
        pltpu.VMEM((K, CW), jnp.int32),
        pltpu.VMEM((K, CW), jnp.int32),
        pltpu.VMEM((CW,), _f32),
        pltpu.VMEM_SHARED((NPAD,), _f32),
        pltpu.VMEM_SHARED((NPAD,), _f32),
        pltpu.SemaphoreType.DMA,
        pltpu.SemaphoreType.DMA,
    ],
)(_deg_body)


def _agg_body(y_hbm, src4d, dst4d, zrows, out_hbm, idx_s, idx_d, rows_v,
              agg_sp, semg0, semg1, semc0, semc1):
    c = lax.axis_index("c")
    s = lax.axis_index("s")
    w = s * NC + c
    pltpu.sync_copy(zrows.at[pl.ds(s * ZR, ZR)], agg_sp.at[pl.ds(s * ZR, ZR)])
    plsc.subcore_barrier()

    semg = (semg0, semg1)
    semc = (semc0, semc1)

    def _step(j, p):
        q = (p + 1) % 2

        @pl.when(j + 1 < WK)
        def _():
            @pl.when(j >= 1)
            def _():
                pltpu.make_async_copy(zrows.at[pl.ds(0, CWA)], rows_v.at[q],
                                      semc[q]).wait()
            pltpu.async_copy(y_hbm.at[idx_s.at[j + 1]], rows_v.at[q], semg[q])

        pltpu.make_async_copy(y_hbm.at[idx_s.at[j]], rows_v.at[p], semg[p]).wait()
        pltpu.async_copy(rows_v.at[p], agg_sp.at[idx_d.at[j]], semc[p], add=True)

    def body(j, carry):
        for p in range(2):
            @pl.when(j % 2 == p)
            def _(p=p):
                _step(j, p)
        return carry

    def window(t, carry):
        pltpu.sync_copy(src4d.at[w, t], idx_s)
        pltpu.sync_copy(dst4d.at[w, t], idx_d)
        pltpu.async_copy(y_hbm.at[idx_s.at[0]], rows_v.at[0], semg0)
        lax.fori_loop(0, WK, body, 0)
        pltpu.make_async_copy(zrows.at[pl.ds(0, CWA)], rows_v.at[(WK - 2) % 2],
                              semc[(WK - 2) % 2]).wait()
        pltpu.make_async_copy(zrows.at[pl.ds(0, CWA)], rows_v.at[(WK - 1) % 2],
                              semc[(WK - 1) % 2]).wait()
        return carry

    lax.fori_loop(0, WT, window, 0)
    plsc.subcore_barrier()
    pltpu.sync_copy(agg_sp.at[pl.ds(s * ZR, ZR)], out_hbm.at[c, pl.ds(s * ZR, ZR)])


_agg_kernel = functools.partial(
    pl.kernel,
    out_type=jax.ShapeDtypeStruct((NC, NPA, D), _f32),
    mesh=_mesh,
    scratch_types=[
        pltpu.VMEM((WK, CWA), jnp.int32),
        pltpu.VMEM((WK, CWA), jnp.int32),
        pltpu.VMEM((2, CWA, D), _f32),
        pltpu.VMEM_SHARED((NPA, D), _f32),
        pltpu.SemaphoreType.DMA,
        pltpu.SemaphoreType.DMA,
        pltpu.SemaphoreType.DMA,
        pltpu.SemaphoreType.DMA,
    ],
)(_agg_body)


def _link_body(p_hbm, q_hbm, srcd, dstd, out_hbm, idx_s, idx_d, rows_v,
               semg0, semg1, semg2, sema0, sema1, sema2, semw0, semw1, semw2):
    c = lax.axis_index("c")
    s = lax.axis_index("s")
    w = s * NC + c
    pltpu.sync_copy(srcd.at[w], idx_s)
    pltpu.sync_copy(dstd.at[w], idx_d)

    semg = (semg0, semg1, semg2)
    sema = (sema0, sema1, sema2)
    semw = (semw0, semw1, semw2)

    pltpu.async_copy(p_hbm.at[idx_s.at[0]], rows_v.at[0], semg0)

    def _step(j, p):
        q = (p + 1) % 3

        @pl.when(j + 1 < K5)
        def _():
            @pl.when(j >= 2)
            def _():
                pltpu.make_async_copy(
                    rows_v.at[q],
                    out_hbm.at[pl.ds(w * EPT5 + (j - 2) * CW, CW)],
                    semw[q]).wait()
            pltpu.async_copy(p_hbm.at[idx_s.at[j + 1]], rows_v.at[q], semg[q])

        pltpu.make_async_copy(p_hbm.at[idx_s.at[j]], rows_v.at[p], semg[p]).wait()
        pltpu.async_copy(q_hbm.at[idx_d.at[j]], rows_v.at[p], sema[p],
                         add=True).wait()
        pltpu.async_copy(rows_v.at[p], out_hbm.at[pl.ds(w * EPT5 + j * CW, CW)],
                         semw[p])

    def body(j, carry):
        for p in range(3):
            @pl.when(j % 3 == p)
            def _(p=p):
                _step(j, p)
        return carry

    lax.fori_loop(0, K5, body, 0)
    pltpu.make_async_copy(rows_v.at[(K5 - 2) % 3],
                          out_hbm.at[pl.ds(w * EPT5 + (K5 - 2) * CW, CW)],
                          semw[(K5 - 2) % 3]).wait()
    pltpu.make_async_copy(rows_v.at[(K5 - 1) % 3],
                          out_hbm.at[pl.ds(w * EPT5 + (K5 - 1) * CW, CW)],
                          semw[(K5 - 1) % 3]).wait()


_link_kernel = functools.partial(
    pl.kernel,
    out_type=jax.ShapeDtypeStruct((ESL, D), _f32),
    mesh=_mesh,
    scratch_types=[
        pltpu.VMEM((K5, CW), jnp.int32),
        pltpu.VMEM((K5, CW), jnp.int32),
        pltpu.VMEM((3, CW, D), _f32),
        pltpu.SemaphoreType.DMA,
        pltpu.SemaphoreType.DMA,
        pltpu.SemaphoreType.DMA,
        pltpu.SemaphoreType.DMA,
        pltpu.SemaphoreType.DMA,
        pltpu.SemaphoreType.DMA,
        pltpu.SemaphoreType.DMA,
        pltpu.SemaphoreType.DMA,
        pltpu.SemaphoreType.DMA,
    ],
)(_link_body)


_RN = 1000
_RE = 8000


def _tc0_body(h_ref, we_ref, be_ref, x_ref):
    x_ref[...] = jnp.dot(h_ref[...], we_ref[...],
                         preferred_element_type=_f32) + be_ref[...]


def _tc1b_body(x_ref, degt_ref, y1_ref, ns_ref, nd_ref):
    dblk = degt_ref[...]
    ns = lax.rsqrt(jnp.maximum(dblk[:, 0:1] + dblk[:, 2:3], 1.0))
    nd = lax.rsqrt(jnp.maximum(dblk[:, 1:2] + dblk[:, 3:4], 1.0))
    y1_ref[...] = x_ref[...] * ns
    ns_ref[...] = ns
    nd_ref[...] = nd


def _tc2_body(aggp_ref, x_ref, ns_ref, nd_ref, w_ref, gb_ref, x2_ref, y2_ref):
    agg = aggp_ref[0] + aggp_ref[1]
    out = jnp.dot(agg * nd_ref[...], w_ref[...], preferred_element_type=_f32)
    r = jnp.maximum(out * gb_ref[0:1, :] + gb_ref[1:2, :], 0.0)
    x2 = x_ref[...] + r
    x2_ref[...] = x2
    y2_ref[...] = x2 * ns_ref[...]


def _tc3a_body(aggp_ref, x_ref, nd_ref, w_ref, gb_ref,
               m0a_ref, m0b_ref, mb0_ref,
               p_ref, qm_ref, x3_ref):
    agg = aggp_ref[0] + aggp_ref[1]
    out = jnp.dot(agg * nd_ref[...], w_ref[...], preferred_element_type=_f32)
    r = jnp.maximum(out * gb_ref[0:1, :] + gb_ref[1:2, :], 0.0)
    x3 = x_ref[...] + r
    x3_ref[...] = x3
    p_ref[...] = jnp.dot(x3, m0a_ref[...], preferred_element_type=_f32) + mb0_ref[...]
    qm_ref[...] = jnp.dot(x3, m0b_ref[...], preferred_element_type=_f32)


def _tc3b_body(x3_ref, q0_ref, qb0_ref, q1_ref, qb1_ref, q2_ref, qb2_ref,
               nscore_ref):
    x3 = x3_ref[...]
    yq = jnp.maximum(jnp.dot(x3, q0_ref[...], preferred_element_type=_f32) + qb0_ref[...], 0.0)
    yq = jnp.maximum(jnp.dot(yq, q1_ref[...], preferred_element_type=_f32) + qb1_ref[...], 0.0)
    nscore_ref[...] = jax.nn.sigmoid(
        jnp.dot(yq, q2_ref[...], preferred_element_type=_f32) + qb2_ref[...])


def _tc4_body(pre_hbm, m1_ref, mb1_ref, m2_ref, mb2_ref, out_ref,
              bufs, sem0, sem1):
    i = pl.program_id(0)
    g = ESL // _RE
    sems = (sem0, sem1)

    @pl.when(i == 0)
    def _():
        pltpu.async_copy(pre_hbm.at[pl.ds(0, _RE)], bufs.at[0], sem0)

    @pl.when(i + 1 < g)
    def _():
        for p in range(2):
            @pl.when((i + 1) % 2 == p)
            def _(p=p):
                pltpu.async_copy(pre_hbm.at[pl.ds((i + 1) * _RE, _RE)],
                                 bufs.at[p], sems[p])

    for p in range(2):
        @pl.when(i % 2 == p)
        def _(p=p):
            pltpu.make_async_copy(pre_hbm.at[pl.ds(i * _RE, _RE)],
                                  bufs.at[p], sems[p]).wait()
            ym = jnp.maximum(bufs[p], 0.0).astype(jnp.bfloat16)
            h1 = jnp.maximum(
                jnp.dot(ym, m1_ref[...], preferred_element_type=_f32)
                + mb1_ref[...], 0.0)
            out_ref[...] = (jnp.dot(h1, m2_ref[...], preferred_element_type=_f32)
                            + mb2_ref[...])


def _tc5_body(z0, z1, z2, z3, z4, out_ref):
    for sl, zr in enumerate((z0, z1, z2, z3, z4)):
        out_ref[pl.ds(sl * (ESL // D), ESL // D), :] = jax.nn.sigmoid(zr[...])


def _row_spec(r, cols):
    return pl.BlockSpec((r, cols), lambda i: (i, 0))


def _full_spec(shape):
    nd = len(shape)
    return pl.BlockSpec(shape, lambda i: (0,) * nd)


def kernel(h, e, W_emb, b_emb, W1, b1, g1, be1, W2, b2, g2, be2,
           M0, mb0, M1, mb1, M2, mb2, Q0, qb0, Q1, qb1, Q2, qb2, edge_index):
    del e
    src = edge_index[0]
    dst = edge_index[1]
    src2d = src.reshape(NW, K, CW)
    dst2d = dst.reshape(NW, K, CW)
    npad_e = EPTA - EPT
    pad_i = jnp.arange(NW * npad_e, dtype=jnp.int32).reshape(NW, npad_e)
    pad_src = (pad_i * 41) % N
    pad_dst = N + (pad_i % (NPA - N))
    src4d = jnp.concatenate([src.reshape(NW, EPT), pad_src], axis=1)
    src4d = src4d.reshape(NW, WT, WK, CWA)
    dst4d = jnp.concatenate([dst.reshape(NW, EPT), pad_dst], axis=1)
    dst4d = dst4d.reshape(NW, WT, WK, CWA)
    zeros1d = jnp.zeros((NPAD,), _f32)
    zrows = jnp.zeros((NPA, D), _f32)
    bn_scale = 1.0 / jnp.sqrt(jnp.float32(1.0 + 1e-5))
    g1e = g1 * bn_scale
    gb1 = jnp.stack([g1e, b1 * g1e + be1])
    g2e = g2 * bn_scale
    gb2 = jnp.stack([g2e, b2 * g2e + be2])

    deg_parts = _deg_kernel(src2d, dst2d, zeros1d)
    deg_t = deg_parts.reshape(4, NPAD).T[:N]

    grid_n = N // _RN
    x = pl.pallas_call(
        _tc0_body,
        grid=(grid_n,),
        in_specs=[_row_spec(_RN, D), _full_spec((D, D)), _full_spec((1, D))],
        out_specs=_row_spec(_RN, D),
        out_shape=jax.ShapeDtypeStruct((N, D), _f32),
    )(h, W_emb, b_emb.reshape(1, D))

    y1, ns, nd = pl.pallas_call(
        _tc1b_body,
        grid=(grid_n,),
        in_specs=[_row_spec(_RN, D), _row_spec(_RN, 4)],
        out_specs=[
            _row_spec(_RN, D), _row_spec(_RN, 1), _row_spec(_RN, 1),
        ],
        out_shape=[
            jax.ShapeDtypeStruct((N, D), _f32),
            jax.ShapeDtypeStruct((N, 1), _f32),
            jax.ShapeDtypeStruct((N, 1), _f32),
        ],
    )(x, deg_t)

    agg1 = _agg_kernel(y1, src4d, dst4d, zrows)
    x2, y2 = pl.pallas_call(
        _tc2_body,
        grid=(grid_n,),
        in_specs=[
            pl.BlockSpec((NC, _RN, D), lambda i: (0, i, 0)),
            _row_spec(_RN, D), _row_spec(_RN, 1), _row_spec(_RN, 1),
            _full_spec((D, D)), _full_spec((2, D)),
        ],
        out_specs=[_row_spec(_RN, D), _row_spec(_RN, D)],
        out_shape=[
            jax.ShapeDtypeStruct((N, D), _f32),
            jax.ShapeDtypeStruct((N, D), _f32),
        ],
    )(agg1, x, ns, nd, W1, gb1)

    agg2 = _agg_kernel(y2, src4d, dst4d, zrows)
    P, Q, x3 = pl.pallas_call(
        _tc3a_body,
        grid=(grid_n,),
        in_specs=[
            pl.BlockSpec((NC, _RN, D), lambda i: (0, i, 0)),
            _row_spec(_RN, D), _row_spec(_RN, 1),
            _full_spec((D, D)), _full_spec((2, D)),
            _full_spec((D, D)), _full_spec((D, D)), _full_spec((1, D)),
        ],
        out_specs=[_row_spec(_RN, D), _row_spec(_RN, D), _row_spec(_RN, D)],
        out_shape=[
            jax.ShapeDtypeStruct((N, D), _f32),
            jax.ShapeDtypeStruct((N, D), _f32),
            jax.ShapeDtypeStruct((N, D), _f32),
        ],
    )(agg2, x2, nd, W2, gb2, M0[:D], M0[D:], mb0.reshape(1, D))

    src5d = src.reshape(NSPL, NW, K5, CW)
    dst5d = dst.reshape(NSPL, NW, K5, CW)
    m1_bf = M1.astype(jnp.bfloat16)
    zs = []
    for sl in range(NSPL):
        pre_sl = _link_kernel(P, Q, src5d[sl], dst5d[sl])
        zs.append(pl.pallas_call(
            _tc4_body,
            grid=(ESL // _RE,),
            in_specs=[
                pl.BlockSpec(memory_space=pl.ANY),
                _full_spec((D, D // 2)), _full_spec((1, D // 2)),
                _full_spec((D // 2, 1)), _full_spec((1, 1)),
            ],
            out_specs=_row_spec(_RE, 1),
            out_shape=jax.ShapeDtypeStruct((ESL, 1), _f32),
            scratch_shapes=[
                pltpu.VMEM((2, _RE, D), _f32),
                pltpu.SemaphoreType.DMA,
                pltpu.SemaphoreType.DMA,
            ],
        )(pre_sl, m1_bf, mb1.reshape(1, -1), M2, mb2.reshape(1, 1)))

    nscore = pl.pallas_call(
        _tc3b_body,
        grid=(grid_n,),
        in_specs=[
            _row_spec(_RN, D),
            _full_spec((D, D // 2)), _full_spec((1, D // 2)),
            _full_spec((D // 2, D // 4)), _full_spec((1, D // 4)),
            _full_spec((D // 4, 1)), _full_spec((1, 1)),
        ],
        out_specs=_row_spec(_RN, 1),
        out_shape=jax.ShapeDtypeStruct((N, 1), _f32),
    )(x3, Q0, qb0.reshape(1, -1), Q1, qb1.reshape(1, -1), Q2, qb2.reshape(1, -1))

    zr = [zz.reshape(ESL // D, D) for zz in zs]
    link = pl.pallas_call(
        _tc5_body,
        grid=(1,),
        in_specs=[_full_spec((ESL // D, D))] * NSPL,
        out_specs=_full_spec((E // D, D)),
        out_shape=jax.ShapeDtypeStruct((E // D, D), _f32),
    )(*zr).reshape(E, 1)

    return (link, nscore)


# speedup vs baseline: 1.0045x; 1.0045x over previous
import functools

import jax
import jax.numpy as jnp
from jax import lax
from jax.experimental import pallas as pl
from jax.experimental.pallas import tpu as pltpu
from jax.experimental.pallas import tpu_sc as plsc

N = 10000
E = 320000
D = 128

NC = 2
NS = 16
NW = NC * NS

CW = 80
EPT = E // NW
K = EPT // CW
NPAD = 10240
ZD = NPAD // NS
CWA = 128
EPTA = 10240
KA = EPTA // CWA
WT = 5
WK = KA // WT
NPA = 10112
ZR = NPA // NS
NSPL = 5
K5 = K // NSPL
ESL = E // NSPL
EPT5 = EPT // NSPL

_f32 = jnp.float32
_mesh = plsc.VectorSubcoreMesh(core_axis_name="c", subcore_axis_name="s")


def _deg_body(src2d, dst2d, zeros1d, out_hbm, idx_s, idx_d, ones_v,
              dout_sp, din_sp, sem, sem2):
    c = lax.axis_index("c")
    s = lax.axis_index("s")
    w = s * NC + c
    for k in range(CW // 16):
        ones_v[pl.ds(k * 16, 16)] = jnp.ones((16,), _f32)
    pltpu.sync_copy(zeros1d.at[pl.ds(s * ZD, ZD)], dout_sp.at[pl.ds(s * ZD, ZD)])
    pltpu.sync_copy(zeros1d.at[pl.ds(s * ZD, ZD)], din_sp.at[pl.ds(s * ZD, ZD)])
    pltpu.sync_copy(src2d.at[w], idx_s)
    pltpu.sync_copy(dst2d.at[w], idx_d)
    plsc.subcore_barrier()

    def body(j, carry):
        pltpu.async_copy(ones_v, dout_sp.at[idx_s.at[j]], sem, add=True)
        pltpu.async_copy(ones_v, din_sp.at[idx_d.at[j]], sem2, add=True)

        @pl.when(j >= 1)
        def _():
            pltpu.make_async_copy(ones_v, dout_sp.at[pl.ds(0, CW)], sem).wait()
            pltpu.make_async_copy(ones_v, din_sp.at[pl.ds(0, CW)], sem2).wait()
        return carry

    lax.fori_loop(0, K, body, 0)
    pltpu.make_async_copy(ones_v, dout_sp.at[pl.ds(0, CW)], sem).wait()
    pltpu.make_async_copy(ones_v, din_sp.at[pl.ds(0, CW)], sem2).wait()
    plsc.subcore_barrier()
    pltpu.sync_copy(dout_sp.at[pl.ds(s * ZD, ZD)], out_hbm.at[c, 0, pl.ds(s * ZD, ZD)])
    pltpu.sync_copy(din_sp.at[pl.ds(s * ZD, ZD)], out_hbm.at[c, 1, pl.ds(s * ZD, ZD)])


_deg_kernel = functools.partial(
    pl.kernel,
    out_type=jax.ShapeDtypeStruct((NC, 2, NPAD), _f32),
    mesh=_mesh,
    scratch_types=[
        pltpu.VMEM((K, CW), jnp.int32),
        pltpu.VMEM((K, CW), jnp.int32),
        pltpu.VMEM((CW,), _f32),
        pltpu.VMEM_SHARED((NPAD,), _f32),
        pltpu.VMEM_SHARED((NPAD,), _f32),
        pltpu.SemaphoreType.DMA,
        pltpu.SemaphoreType.DMA,
    ],
)(_deg_body)


def _agg_body(y_hbm, src4d, dst4d, zrows, out_hbm, idx_s, idx_d, rows_v,
              agg_sp, semg0, semg1, semc0, semc1):
    c = lax.axis_index("c")
    s = lax.axis_index("s")
    w = s * NC + c
    pltpu.sync_copy(zrows.at[pl.ds(s * ZR, ZR)], agg_sp.at[pl.ds(s * ZR, ZR)])
    plsc.subcore_barrier()

    semg = (semg0, semg1)
    semc = (semc0, semc1)

    def _step(j, p):
        q = (p + 1) % 2

        @pl.when(j + 1 < WK)
        def _():
            @pl.when(j >= 1)
            def _():
                pltpu.make_async_copy(zrows.at[pl.ds(0, CWA)], rows_v.at[q],
                                      semc[q]).wait()
            pltpu.async_copy(y_hbm.at[idx_s.at[j + 1]], rows_v.at[q], semg[q])

        pltpu.make_async_copy(y_hbm.at[idx_s.at[j]], rows_v.at[p], semg[p]).wait()
        pltpu.async_copy(rows_v.at[p], agg_sp.at[idx_d.at[j]], semc[p], add=True)

    def body(j, carry):
        for p in range(2):
            @pl.when(j % 2 == p)
            def _(p=p):
                _step(j, p)
        return carry

    def window(t, carry):
        pltpu.sync_copy(src4d.at[w, t], idx_s)
        pltpu.sync_copy(dst4d.at[w, t], idx_d)
        pltpu.async_copy(y_hbm.at[idx_s.at[0]], rows_v.at[0], semg0)
        lax.fori_loop(0, WK, body, 0)
        pltpu.make_async_copy(zrows.at[pl.ds(0, CWA)], rows_v.at[(WK - 2) % 2],
                              semc[(WK - 2) % 2]).wait()
        pltpu.make_async_copy(zrows.at[pl.ds(0, CWA)], rows_v.at[(WK - 1) % 2],
                              semc[(WK - 1) % 2]).wait()
        return carry

    lax.fori_loop(0, WT, window, 0)
    plsc.subcore_barrier()
    pltpu.sync_copy(agg_sp.at[pl.ds(s * ZR, ZR)], out_hbm.at[c, pl.ds(s * ZR, ZR)])


_agg_kernel = functools.partial(
    pl.kernel,
    out_type=jax.ShapeDtypeStruct((NC, NPA, D), _f32),
    mesh=_mesh,
    scratch_types=[
        pltpu.VMEM((WK, CWA), jnp.int32),
        pltpu.VMEM((WK, CWA), jnp.int32),
        pltpu.VMEM((2, CWA, D), _f32),
        pltpu.VMEM_SHARED((NPA, D), _f32),
        pltpu.SemaphoreType.DMA,
        pltpu.SemaphoreType.DMA,
        pltpu.SemaphoreType.DMA,
        pltpu.SemaphoreType.DMA,
    ],
)(_agg_body)


def _link_body(p_hbm, q_hbm, srcd, dstd, out_hbm, idx_s, idx_d, rows_v,
               semg0, semg1, semg2, sema0, sema1, sema2, semw0, semw1, semw2):
    c = lax.axis_index("c")
    s = lax.axis_index("s")
    w = s * NC + c
    pltpu.sync_copy(srcd.at[w], idx_s)
    pltpu.sync_copy(dstd.at[w], idx_d)

    semg = (semg0, semg1, semg2)
    sema = (sema0, sema1, sema2)
    semw = (semw0, semw1, semw2)

    pltpu.async_copy(p_hbm.at[idx_s.at[0]], rows_v.at[0], semg0)

    def _step(j, p):
        q = (p + 1) % 3

        @pl.when(j + 1 < K5)
        def _():
            @pl.when(j >= 2)
            def _():
                pltpu.make_async_copy(
                    rows_v.at[q],
                    out_hbm.at[pl.ds(w * EPT5 + (j - 2) * CW, CW)],
                    semw[q]).wait()
            pltpu.async_copy(p_hbm.at[idx_s.at[j + 1]], rows_v.at[q], semg[q])

        pltpu.make_async_copy(p_hbm.at[idx_s.at[j]], rows_v.at[p], semg[p]).wait()
        pltpu.async_copy(q_hbm.at[idx_d.at[j]], rows_v.at[p], sema[p],
                         add=True).wait()
        pltpu.async_copy(rows_v.at[p], out_hbm.at[pl.ds(w * EPT5 + j * CW, CW)],
                         semw[p])

    def body(j, carry):
        for p in range(3):
            @pl.when(j % 3 == p)
            def _(p=p):
                _step(j, p)
        return carry

    lax.fori_loop(0, K5, body, 0)
    pltpu.make_async_copy(rows_v.at[(K5 - 2) % 3],
                          out_hbm.at[pl.ds(w * EPT5 + (K5 - 2) * CW, CW)],
                          semw[(K5 - 2) % 3]).wait()
    pltpu.make_async_copy(rows_v.at[(K5 - 1) % 3],
                          out_hbm.at[pl.ds(w * EPT5 + (K5 - 1) * CW, CW)],
                          semw[(K5 - 1) % 3]).wait()


_link_kernel = functools.partial(
    pl.kernel,
    out_type=jax.ShapeDtypeStruct((ESL, D), _f32),
    mesh=_mesh,
    scratch_types=[
        pltpu.VMEM((K5, CW), jnp.int32),
        pltpu.VMEM((K5, CW), jnp.int32),
        pltpu.VMEM((3, CW, D), _f32),
        pltpu.SemaphoreType.DMA,
        pltpu.SemaphoreType.DMA,
        pltpu.SemaphoreType.DMA,
        pltpu.SemaphoreType.DMA,
        pltpu.SemaphoreType.DMA,
        pltpu.SemaphoreType.DMA,
        pltpu.SemaphoreType.DMA,
        pltpu.SemaphoreType.DMA,
        pltpu.SemaphoreType.DMA,
    ],
)(_link_body)


_RN = 1000
_RE = 8000


def _tc0_body(h_ref, we_ref, be_ref, x_ref):
    x_ref[...] = jnp.dot(h_ref[...], we_ref[...],
                         preferred_element_type=_f32) + be_ref[...]


def _tc1b_body(x_ref, degt_ref, y1_ref, ns_ref, nd_ref):
    dblk = degt_ref[...]
    ns = lax.rsqrt(jnp.maximum(dblk[:, 0:1] + dblk[:, 2:3], 1.0))
    nd = lax.rsqrt(jnp.maximum(dblk[:, 1:2] + dblk[:, 3:4], 1.0))
    y1_ref[...] = x_ref[...] * ns
    ns_ref[...] = ns
    nd_ref[...] = nd


def _tc2_body(aggp_ref, x_ref, ns_ref, nd_ref, w_ref, gb_ref, x2_ref, y2_ref):
    agg = aggp_ref[0] + aggp_ref[1]
    out = jnp.dot(agg * nd_ref[...], w_ref[...], preferred_element_type=_f32)
    r = jnp.maximum(out * gb_ref[0:1, :] + gb_ref[1:2, :], 0.0)
    x2 = x_ref[...] + r
    x2_ref[...] = x2
    y2_ref[...] = x2 * ns_ref[...]


def _tc3a_body(aggp_ref, x_ref, nd_ref, w_ref, gb_ref,
               m0a_ref, m0b_ref, mb0_ref,
               p_ref, qm_ref, x3_ref):
    agg = aggp_ref[0] + aggp_ref[1]
    out = jnp.dot(agg * nd_ref[...], w_ref[...], preferred_element_type=_f32)
    r = jnp.maximum(out * gb_ref[0:1, :] + gb_ref[1:2, :], 0.0)
    x3 = x_ref[...] + r
    x3_ref[...] = x3
    p_ref[...] = jnp.dot(x3, m0a_ref[...], preferred_element_type=_f32) + mb0_ref[...]
    qm_ref[...] = jnp.dot(x3, m0b_ref[...], preferred_element_type=_f32)


def _tc3b_body(x3_ref, q0_ref, qb0_ref, q1_ref, qb1_ref, q2_ref, qb2_ref,
               nscore_ref):
    x3 = x3_ref[...]
    yq = jnp.maximum(jnp.dot(x3, q0_ref[...], preferred_element_type=_f32) + qb0_ref[...], 0.0)
    yq = jnp.maximum(jnp.dot(yq, q1_ref[...], preferred_element_type=_f32) + qb1_ref[...], 0.0)
    nscore_ref[...] = jax.nn.sigmoid(
        jnp.dot(yq, q2_ref[...], preferred_element_type=_f32) + qb2_ref[...])


def _tc4_body(pre_ref, m1_ref, mb1_ref, m2_ref, mb2_ref, out_ref):
    ym = jnp.maximum(pre_ref[...], 0.0).astype(jnp.bfloat16)
    h1 = jnp.maximum(
        jnp.dot(ym, m1_ref[...], preferred_element_type=_f32) + mb1_ref[...], 0.0)
    out_ref[...] = jnp.dot(h1, m2_ref[...], preferred_element_type=_f32) + mb2_ref[...]


def _tc5_body(z0, z1, z2, z3, z4, out_ref):
    for sl, zr in enumerate((z0, z1, z2, z3, z4)):
        out_ref[pl.ds(sl * (ESL // D), ESL // D), :] = jax.nn.sigmoid(zr[...])


def _row_spec(r, cols):
    return pl.BlockSpec((r, cols), lambda i: (i, 0))


def _full_spec(shape):
    nd = len(shape)
    return pl.BlockSpec(shape, lambda i: (0,) * nd)


def kernel(h, e, W_emb, b_emb, W1, b1, g1, be1, W2, b2, g2, be2,
           M0, mb0, M1, mb1, M2, mb2, Q0, qb0, Q1, qb1, Q2, qb2, edge_index):
    del e
    src = edge_index[0]
    dst = edge_index[1]
    src2d = src.reshape(NW, K, CW)
    dst2d = dst.reshape(NW, K, CW)
    npad_e = EPTA - EPT
    pad_i = jnp.arange(NW * npad_e, dtype=jnp.int32).reshape(NW, npad_e)
    pad_src = (pad_i * 41) % N
    pad_dst = N + (pad_i % (NPA - N))
    src4d = jnp.concatenate([src.reshape(NW, EPT), pad_src], axis=1)
    src4d = src4d.reshape(NW, WT, WK, CWA)
    dst4d = jnp.concatenate([dst.reshape(NW, EPT), pad_dst], axis=1)
    dst4d = dst4d.reshape(NW, WT, WK, CWA)
    zeros1d = jnp.zeros((NPAD,), _f32)
    zrows = jnp.zeros((NPA, D), _f32)
    bn_scale = 1.0 / jnp.sqrt(jnp.float32(1.0 + 1e-5))
    g1e = g1 * bn_scale
    gb1 = jnp.stack([g1e, b1 * g1e + be1])
    g2e = g2 * bn_scale
    gb2 = jnp.stack([g2e, b2 * g2e + be2])

    deg_parts = _deg_kernel(src2d, dst2d, zeros1d)
    deg_t = deg_parts.reshape(4, NPAD).T[:N]

    grid_n = N // _RN
    x = pl.pallas_call(
        _tc0_body,
        grid=(grid_n,),
        in_specs=[_row_spec(_RN, D), _full_spec((D, D)), _full_spec((1, D))],
        out_specs=_row_spec(_RN, D),
        out_shape=jax.ShapeDtypeStruct((N, D), _f32),
    )(h, W_emb, b_emb.reshape(1, D))

    y1, ns, nd = pl.pallas_call(
        _tc1b_body,
        grid=(grid_n,),
        in_specs=[_row_spec(_RN, D), _row_spec(_RN, 4)],
        out_specs=[
            _row_spec(_RN, D), _row_spec(_RN, 1), _row_spec(_RN, 1),
        ],
        out_shape=[
            jax.ShapeDtypeStruct((N, D), _f32),
            jax.ShapeDtypeStruct((N, 1), _f32),
            jax.ShapeDtypeStruct((N, 1), _f32),
        ],
    )(x, deg_t)

    agg1 = _agg_kernel(y1, src4d, dst4d, zrows)
    x2, y2 = pl.pallas_call(
        _tc2_body,
        grid=(grid_n,),
        in_specs=[
            pl.BlockSpec((NC, _RN, D), lambda i: (0, i, 0)),
            _row_spec(_RN, D), _row_spec(_RN, 1), _row_spec(_RN, 1),
            _full_spec((D, D)), _full_spec((2, D)),
        ],
        out_specs=[_row_spec(_RN, D), _row_spec(_RN, D)],
        out_shape=[
            jax.ShapeDtypeStruct((N, D), _f32),
            jax.ShapeDtypeStruct((N, D), _f32),
        ],
    )(agg1, x, ns, nd, W1, gb1)

    agg2 = _agg_kernel(y2, src4d, dst4d, zrows)
    P, Q, x3 = pl.pallas_call(
        _tc3a_body,
        grid=(grid_n,),
        in_specs=[
            pl.BlockSpec((NC, _RN, D), lambda i: (0, i, 0)),
            _row_spec(_RN, D), _row_spec(_RN, 1),
            _full_spec((D, D)), _full_spec((2, D)),
            _full_spec((D, D)), _full_spec((D, D)), _full_spec((1, D)),
        ],
        out_specs=[_row_spec(_RN, D), _row_spec(_RN, D), _row_spec(_RN, D)],
        out_shape=[
            jax.ShapeDtypeStruct((N, D), _f32),
            jax.ShapeDtypeStruct((N, D), _f32),
            jax.ShapeDtypeStruct((N, D), _f32),
        ],
    )(agg2, x2, nd, W2, gb2, M0[:D], M0[D:], mb0.reshape(1, D))

    src5d = src.reshape(NSPL, NW, K5, CW)
    dst5d = dst.reshape(NSPL, NW, K5, CW)
    m1_bf = M1.astype(jnp.bfloat16)
    zs = []
    for sl in range(NSPL):
        pre_sl = _link_kernel(P, Q, src5d[sl], dst5d[sl])
        zs.append(pl.pallas_call(
            _tc4_body,
            grid=(ESL // _RE,),
            in_specs=[
                _row_spec(_RE, D),
                _full_spec((D, D // 2)), _full_spec((1, D // 2)),
                _full_spec((D // 2, 1)), _full_spec((1, 1)),
            ],
            out_specs=_row_spec(_RE, 1),
            out_shape=jax.ShapeDtypeStruct((ESL, 1), _f32),
        )(pre_sl, m1_bf, mb1.reshape(1, -1), M2, mb2.reshape(1, 1)))

    nscore = pl.pallas_call(
        _tc3b_body,
        grid=(grid_n,),
        in_specs=[
            _row_spec(_RN, D),
            _full_spec((D, D // 2)), _full_spec((1, D // 2)),
            _full_spec((D // 2, D // 4)), _full_spec((1, D // 4)),
            _full_spec((D // 4, 1)), _full_spec((1, 1)),
        ],
        out_specs=_row_spec(_RN, 1),
        out_shape=jax.ShapeDtypeStruct((N, 1), _f32),
    )(x3, Q0, qb0.reshape(1, -1), Q1, qb1.reshape(1, -1), Q2, qb2.reshape(1, -1))

    zr = [zz.reshape(ESL // D, D) for zz in zs]
    link = pl.pallas_call(
        _tc5_body,
        grid=(1,),
        in_specs=[_full_spec((ESL // D, D))] * NSPL,
        out_specs=_full_spec((E // D, D)),
        out_shape=jax.ShapeDtypeStruct((E // D, D), _f32),
    )(*zr).reshape(E, 1)

    return (link, nscore)


# speedup vs baseline: 1.0228x; 1.0182x over previous
import functools

import jax
import jax.numpy as jnp
from jax import lax
from jax.experimental import pallas as pl
from jax.experimental.pallas import tpu as pltpu
from jax.experimental.pallas import tpu_sc as plsc

N = 10000
E = 320000
D = 128

NC = 2
NS = 16
NW = NC * NS

CW = 80
EPT = E // NW
K = EPT // CW
NPAD = 10240
ZD = NPAD // NS
CWA = 64
EPTA = 10240
KA = EPTA // CWA
WT = 5
WK = KA // WT
NPA = 10112
ZR = NPA // NS
NSPL = 5
K5 = K // NSPL
ESL = E // NSPL
EPT5 = EPT // NSPL

_f32 = jnp.float32
_mesh = plsc.VectorSubcoreMesh(core_axis_name="c", subcore_axis_name="s")


def _deg_body(src2d, dst2d, zeros1d, out_hbm, idx_s, idx_d, ones_v,
              dout_sp, din_sp, sem, sem2):
    c = lax.axis_index("c")
    s = lax.axis_index("s")
    w = s * NC + c
    for k in range(CW // 16):
        ones_v[pl.ds(k * 16, 16)] = jnp.ones((16,), _f32)
    pltpu.sync_copy(zeros1d.at[pl.ds(s * ZD, ZD)], dout_sp.at[pl.ds(s * ZD, ZD)])
    pltpu.sync_copy(zeros1d.at[pl.ds(s * ZD, ZD)], din_sp.at[pl.ds(s * ZD, ZD)])
    pltpu.sync_copy(src2d.at[w], idx_s)
    pltpu.sync_copy(dst2d.at[w], idx_d)
    plsc.subcore_barrier()

    def body(j, carry):
        pltpu.async_copy(ones_v, dout_sp.at[idx_s.at[j]], sem, add=True)
        pltpu.async_copy(ones_v, din_sp.at[idx_d.at[j]], sem2, add=True)

        @pl.when(j >= 1)
        def _():
            pltpu.make_async_copy(ones_v, dout_sp.at[pl.ds(0, CW)], sem).wait()
            pltpu.make_async_copy(ones_v, din_sp.at[pl.ds(0, CW)], sem2).wait()
        return carry

    lax.fori_loop(0, K, body, 0)
    pltpu.make_async_copy(ones_v, dout_sp.at[pl.ds(0, CW)], sem).wait()
    pltpu.make_async_copy(ones_v, din_sp.at[pl.ds(0, CW)], sem2).wait()
    plsc.subcore_barrier()
    pltpu.sync_copy(dout_sp.at[pl.ds(s * ZD, ZD)], out_hbm.at[c, 0, pl.ds(s * ZD, ZD)])
    pltpu.sync_copy(din_sp.at[pl.ds(s * ZD, ZD)], out_hbm.at[c, 1, pl.ds(s * ZD, ZD)])


_deg_kernel = functools.partial(
    pl.kernel,
    out_type=jax.ShapeDtypeStruct((NC, 2, NPAD), _f32),
    mesh=_mesh,
    scratch_types=[
        pltpu.VMEM((K, CW), jnp.int32),
        pltpu.VMEM((K, CW), jnp.int32),
        pltpu.VMEM((CW,), _f32),
        pltpu.VMEM_SHARED((NPAD,), _f32),
        pltpu.VMEM_SHARED((NPAD,), _f32),
        pltpu.SemaphoreType.DMA,
        pltpu.SemaphoreType.DMA,
    ],
)(_deg_body)


def _agg_body(y_hbm, src4d, dst4d, zrows, out_hbm, idx_s, idx_d, rows_v,
              agg_sp, semg0, semg1, semg2, semg3, semc0, semc1, semc2, semc3):
    c = lax.axis_index("c")
    s = lax.axis_index("s")
    w = s * NC + c
    pltpu.sync_copy(zrows.at[pl.ds(s * ZR, ZR)], agg_sp.at[pl.ds(s * ZR, ZR)])
    plsc.subcore_barrier()

    semg = (semg0, semg1, semg2, semg3)
    semc = (semc0, semc1, semc2, semc3)

    def _drain(b):
        pltpu.make_async_copy(zrows.at[pl.ds(0, CWA)], rows_v.at[b],
                              semc[b]).wait()

    def _step(j, p):
        q3 = (p + 3) % 4

        @pl.when(j + 3 < WK)
        def _():
            @pl.when(j >= 1)
            def _():
                _drain(q3)
            pltpu.async_copy(y_hbm.at[idx_s.at[j + 3]], rows_v.at[q3],
                             semg[q3])

        pltpu.make_async_copy(y_hbm.at[idx_s.at[j]], rows_v.at[p],
                              semg[p]).wait()
        pltpu.async_copy(rows_v.at[p], agg_sp.at[idx_d.at[j]], semc[p],
                         add=True)

    def body(j, carry):
        for p in range(4):
            @pl.when(j % 4 == p)
            def _(p=p):
                _step(j, p)
        return carry

    def window(t, carry):
        pltpu.sync_copy(src4d.at[w, t], idx_s)
        pltpu.sync_copy(dst4d.at[w, t], idx_d)
        for jj in range(3):
            pltpu.async_copy(y_hbm.at[idx_s.at[jj]], rows_v.at[jj], semg[jj])
        lax.fori_loop(0, WK, body, 0)
        for r in range(WK - 4, WK):
            _drain(r % 4)
        return carry

    lax.fori_loop(0, WT, window, 0)
    plsc.subcore_barrier()
    pltpu.sync_copy(agg_sp.at[pl.ds(s * ZR, ZR)], out_hbm.at[c, pl.ds(s * ZR, ZR)])


_agg_kernel = functools.partial(
    pl.kernel,
    out_type=jax.ShapeDtypeStruct((NC, NPA, D), _f32),
    mesh=_mesh,
    scratch_types=[
        pltpu.VMEM((WK, CWA), jnp.int32),
        pltpu.VMEM((WK, CWA), jnp.int32),
        pltpu.VMEM((4, CWA, D), _f32),
        pltpu.VMEM_SHARED((NPA, D), _f32),
        pltpu.SemaphoreType.DMA,
        pltpu.SemaphoreType.DMA,
        pltpu.SemaphoreType.DMA,
        pltpu.SemaphoreType.DMA,
        pltpu.SemaphoreType.DMA,
        pltpu.SemaphoreType.DMA,
        pltpu.SemaphoreType.DMA,
        pltpu.SemaphoreType.DMA,
    ],
)(_agg_body)


def _link_body(p_hbm, q_hbm, srcd, dstd, out_hbm, idx_s, idx_d, rows_v,
               semg0, semg1, semg2, sema0, sema1, sema2, semw0, semw1, semw2):
    c = lax.axis_index("c")
    s = lax.axis_index("s")
    w = s * NC + c
    pltpu.sync_copy(srcd.at[w], idx_s)
    pltpu.sync_copy(dstd.at[w], idx_d)

    semg = (semg0, semg1, semg2)
    sema = (sema0, sema1, sema2)
    semw = (semw0, semw1, semw2)

    pltpu.async_copy(p_hbm.at[idx_s.at[0]], rows_v.at[0], semg0)

    def _step(j, p):
        q = (p + 1) % 3

        @pl.when(j + 1 < K5)
        def _():
            @pl.when(j >= 2)
            def _():
                pltpu.make_async_copy(
                    rows_v.at[q],
                    out_hbm.at[pl.ds(w * EPT5 + (j - 2) * CW, CW)],
                    semw[q]).wait()
            pltpu.async_copy(p_hbm.at[idx_s.at[j + 1]], rows_v.at[q], semg[q])

        pltpu.make_async_copy(p_hbm.at[idx_s.at[j]], rows_v.at[p], semg[p]).wait()
        pltpu.async_copy(q_hbm.at[idx_d.at[j]], rows_v.at[p], sema[p],
                         add=True).wait()
        pltpu.async_copy(rows_v.at[p], out_hbm.at[pl.ds(w * EPT5 + j * CW, CW)],
                         semw[p])

    def body(j, carry):
        for p in range(3):
            @pl.when(j % 3 == p)
            def _(p=p):
                _step(j, p)
        return carry

    lax.fori_loop(0, K5, body, 0)
    pltpu.make_async_copy(rows_v.at[(K5 - 2) % 3],
                          out_hbm.at[pl.ds(w * EPT5 + (K5 - 2) * CW, CW)],
                          semw[(K5 - 2) % 3]).wait()
    pltpu.make_async_copy(rows_v.at[(K5 - 1) % 3],
                          out_hbm.at[pl.ds(w * EPT5 + (K5 - 1) * CW, CW)],
                          semw[(K5 - 1) % 3]).wait()


_link_kernel = functools.partial(
    pl.kernel,
    out_type=jax.ShapeDtypeStruct((ESL, D), _f32),
    mesh=_mesh,
    scratch_types=[
        pltpu.VMEM((K5, CW), jnp.int32),
        pltpu.VMEM((K5, CW), jnp.int32),
        pltpu.VMEM((3, CW, D), _f32),
        pltpu.SemaphoreType.DMA,
        pltpu.SemaphoreType.DMA,
        pltpu.SemaphoreType.DMA,
        pltpu.SemaphoreType.DMA,
        pltpu.SemaphoreType.DMA,
        pltpu.SemaphoreType.DMA,
        pltpu.SemaphoreType.DMA,
        pltpu.SemaphoreType.DMA,
        pltpu.SemaphoreType.DMA,
    ],
)(_link_body)


_RN = 1000
_RE = 8000


def _tc0_body(h_ref, we_ref, be_ref, x_ref):
    x_ref[...] = jnp.dot(h_ref[...], we_ref[...],
                         preferred_element_type=_f32) + be_ref[...]


def _tc1b_body(x_ref, degt_ref, y1_ref, ns_ref, nd_ref):
    dblk = degt_ref[...]
    ns = lax.rsqrt(jnp.maximum(dblk[:, 0:1] + dblk[:, 2:3], 1.0))
    nd = lax.rsqrt(jnp.maximum(dblk[:, 1:2] + dblk[:, 3:4], 1.0))
    y1_ref[...] = x_ref[...] * ns
    ns_ref[...] = ns
    nd_ref[...] = nd


def _tc2_body(aggp_ref, x_ref, ns_ref, nd_ref, w_ref, gb_ref, x2_ref, y2_ref):
    agg = aggp_ref[0] + aggp_ref[1]
    out = jnp.dot(agg * nd_ref[...], w_ref[...], preferred_element_type=_f32)
    r = jnp.maximum(out * gb_ref[0:1, :] + gb_ref[1:2, :], 0.0)
    x2 = x_ref[...] + r
    x2_ref[...] = x2
    y2_ref[...] = x2 * ns_ref[...]


def _tc3a_body(aggp_ref, x_ref, nd_ref, w_ref, gb_ref,
               m0a_ref, m0b_ref, mb0_ref,
               p_ref, qm_ref, x3_ref):
    agg = aggp_ref[0] + aggp_ref[1]
    out = jnp.dot(agg * nd_ref[...], w_ref[...], preferred_element_type=_f32)
    r = jnp.maximum(out * gb_ref[0:1, :] + gb_ref[1:2, :], 0.0)
    x3 = x_ref[...] + r
    x3_ref[...] = x3
    p_ref[...] = jnp.dot(x3, m0a_ref[...], preferred_element_type=_f32) + mb0_ref[...]
    qm_ref[...] = jnp.dot(x3, m0b_ref[...], preferred_element_type=_f32)


def _tc3b_body(x3_ref, q0_ref, qb0_ref, q1_ref, qb1_ref, q2_ref, qb2_ref,
               nscore_ref):
    x3 = x3_ref[...]
    yq = jnp.maximum(jnp.dot(x3, q0_ref[...], preferred_element_type=_f32) + qb0_ref[...], 0.0)
    yq = jnp.maximum(jnp.dot(yq, q1_ref[...], preferred_element_type=_f32) + qb1_ref[...], 0.0)
    nscore_ref[...] = jax.nn.sigmoid(
        jnp.dot(yq, q2_ref[...], preferred_element_type=_f32) + qb2_ref[...])


def _tc4_body(pre_ref, m1_ref, mb1_ref, m2_ref, mb2_ref, out_ref):
    ym = jnp.maximum(pre_ref[...], 0.0).astype(jnp.bfloat16)
    h1 = jnp.maximum(
        jnp.dot(ym, m1_ref[...], preferred_element_type=_f32) + mb1_ref[...], 0.0)
    out_ref[...] = jnp.dot(h1, m2_ref[...], preferred_element_type=_f32) + mb2_ref[...]


def _tc5_body(z0, z1, z2, z3, z4, out_ref):
    for sl, zr in enumerate((z0, z1, z2, z3, z4)):
        out_ref[pl.ds(sl * (ESL // D), ESL // D), :] = jax.nn.sigmoid(zr[...])


def _row_spec(r, cols):
    return pl.BlockSpec((r, cols), lambda i: (i, 0))


def _full_spec(shape):
    nd = len(shape)
    return pl.BlockSpec(shape, lambda i: (0,) * nd)


def kernel(h, e, W_emb, b_emb, W1, b1, g1, be1, W2, b2, g2, be2,
           M0, mb0, M1, mb1, M2, mb2, Q0, qb0, Q1, qb1, Q2, qb2, edge_index):
    del e
    src = edge_index[0]
    dst = edge_index[1]
    src2d = src.reshape(NW, K, CW)
    dst2d = dst.reshape(NW, K, CW)
    npad_e = EPTA - EPT
    pad_i = jnp.arange(NW * npad_e, dtype=jnp.int32).reshape(NW, npad_e)
    pad_src = (pad_i * 41) % N
    pad_dst = N + (pad_i % (NPA - N))
    src4d = jnp.concatenate([src.reshape(NW, EPT), pad_src], axis=1)
    src4d = src4d.reshape(NW, WT, WK, CWA)
    dst4d = jnp.concatenate([dst.reshape(NW, EPT), pad_dst], axis=1)
    dst4d = dst4d.reshape(NW, WT, WK, CWA)
    zeros1d = jnp.zeros((NPAD,), _f32)
    zrows = jnp.zeros((NPA, D), _f32)
    bn_scale = 1.0 / jnp.sqrt(jnp.float32(1.0 + 1e-5))
    g1e = g1 * bn_scale
    gb1 = jnp.stack([g1e, b1 * g1e + be1])
    g2e = g2 * bn_scale
    gb2 = jnp.stack([g2e, b2 * g2e + be2])

    deg_parts = _deg_kernel(src2d, dst2d, zeros1d)
    deg_t = deg_parts.reshape(4, NPAD).T[:N]

    grid_n = N // _RN
    x = pl.pallas_call(
        _tc0_body,
        grid=(grid_n,),
        in_specs=[_row_spec(_RN, D), _full_spec((D, D)), _full_spec((1, D))],
        out_specs=_row_spec(_RN, D),
        out_shape=jax.ShapeDtypeStruct((N, D), _f32),
    )(h, W_emb, b_emb.reshape(1, D))

    y1, ns, nd = pl.pallas_call(
        _tc1b_body,
        grid=(grid_n,),
        in_specs=[_row_spec(_RN, D), _row_spec(_RN, 4)],
        out_specs=[
            _row_spec(_RN, D), _row_spec(_RN, 1), _row_spec(_RN, 1),
        ],
        out_shape=[
            jax.ShapeDtypeStruct((N, D), _f32),
            jax.ShapeDtypeStruct((N, 1), _f32),
            jax.ShapeDtypeStruct((N, 1), _f32),
        ],
    )(x, deg_t)

    agg1 = _agg_kernel(y1, src4d, dst4d, zrows)
    x2, y2 = pl.pallas_call(
        _tc2_body,
        grid=(grid_n,),
        in_specs=[
            pl.BlockSpec((NC, _RN, D), lambda i: (0, i, 0)),
            _row_spec(_RN, D), _row_spec(_RN, 1), _row_spec(_RN, 1),
            _full_spec((D, D)), _full_spec((2, D)),
        ],
        out_specs=[_row_spec(_RN, D), _row_spec(_RN, D)],
        out_shape=[
            jax.ShapeDtypeStruct((N, D), _f32),
            jax.ShapeDtypeStruct((N, D), _f32),
        ],
    )(agg1, x, ns, nd, W1, gb1)

    agg2 = _agg_kernel(y2, src4d, dst4d, zrows)
    P, Q, x3 = pl.pallas_call(
        _tc3a_body,
        grid=(grid_n,),
        in_specs=[
            pl.BlockSpec((NC, _RN, D), lambda i: (0, i, 0)),
            _row_spec(_RN, D), _row_spec(_RN, 1),
            _full_spec((D, D)), _full_spec((2, D)),
            _full_spec((D, D)), _full_spec((D, D)), _full_spec((1, D)),
        ],
        out_specs=[_row_spec(_RN, D), _row_spec(_RN, D), _row_spec(_RN, D)],
        out_shape=[
            jax.ShapeDtypeStruct((N, D), _f32),
            jax.ShapeDtypeStruct((N, D), _f32),
            jax.ShapeDtypeStruct((N, D), _f32),
        ],
    )(agg2, x2, nd, W2, gb2, M0[:D], M0[D:], mb0.reshape(1, D))

    src5d = src.reshape(NSPL, NW, K5, CW)
    dst5d = dst.reshape(NSPL, NW, K5, CW)
    m1_bf = M1.astype(jnp.bfloat16)
    zs = []
    for sl in range(NSPL):
        pre_sl = _link_kernel(P, Q, src5d[sl], dst5d[sl])
        zs.append(pl.pallas_call(
            _tc4_body,
            grid=(ESL // _RE,),
            in_specs=[
                _row_spec(_RE, D),
                _full_spec((D, D // 2)), _full_spec((1, D // 2)),
                _full_spec((D // 2, 1)), _full_spec((1, 1)),
            ],
            out_specs=_row_spec(_RE, 1),
            out_shape=jax.ShapeDtypeStruct((ESL, 1), _f32),
        )(pre_sl, m1_bf, mb1.reshape(1, -1), M2, mb2.reshape(1, 1)))

    nscore = pl.pallas_call(
        _tc3b_body,
        grid=(grid_n,),
        in_specs=[
            _row_spec(_RN, D),
            _full_spec((D, D // 2)), _full_spec((1, D // 2)),
            _full_spec((D // 2, D // 4)), _full_spec((1, D // 4)),
            _full_spec((D // 4, 1)), _full_spec((1, 1)),
        ],
        out_specs=_row_spec(_RN, 1),
        out_shape=jax.ShapeDtypeStruct((N, 1), _f32),
    )(x3, Q0, qb0.reshape(1, -1), Q1, qb1.reshape(1, -1), Q2, qb2.reshape(1, -1))

    zr = [zz.reshape(ESL // D, D) for zz in zs]
    link = pl.pallas_call(
        _tc5_body,
        grid=(1,),
        in_specs=[_full_spec((ESL // D, D))] * NSPL,
        out_specs=_full_spec((E // D, D)),
        out_shape=jax.ShapeDtypeStruct((E // D, D), _f32),
    )(*zr).reshape(E, 1)

    return (link, nscore)


# speedup vs baseline: 1.1771x; 1.1509x over previous
import functools

import jax
import jax.numpy as jnp
from jax import lax
from jax.experimental import pallas as pl
from jax.experimental.pallas import tpu as pltpu
from jax.experimental.pallas import tpu_sc as plsc

N = 10000
E = 320000
D = 128

NC = 2
NS = 16
NW = NC * NS

CW = 80
EPT = E // NW
K = EPT // CW
NPAD = 10240
ZD = NPAD // NS
CWA = 64
EPTA = 10240
KA = EPTA // CWA
WT = 5
WK = KA // WT
NPA = 10112
ZR = NPA // NS
NSPL = 5
K5 = K // NSPL
ESL = E // NSPL
EPT5 = EPT // NSPL

_f32 = jnp.float32
_mesh = plsc.VectorSubcoreMesh(core_axis_name="c", subcore_axis_name="s")


def _deg_body(src2d, dst2d, zeros1d, out_hbm, idx_s, idx_d, ones_v,
              dout_sp, din_sp, sem, sem2):
    c = lax.axis_index("c")
    s = lax.axis_index("s")
    w = s * NC + c
    for k in range(CW // 16):
        ones_v[pl.ds(k * 16, 16)] = jnp.ones((16,), _f32)
    pltpu.sync_copy(zeros1d.at[pl.ds(s * ZD, ZD)], dout_sp.at[pl.ds(s * ZD, ZD)])
    pltpu.sync_copy(zeros1d.at[pl.ds(s * ZD, ZD)], din_sp.at[pl.ds(s * ZD, ZD)])
    pltpu.sync_copy(src2d.at[w], idx_s)
    pltpu.sync_copy(dst2d.at[w], idx_d)
    plsc.subcore_barrier()

    def body(j, carry):
        pltpu.async_copy(ones_v, dout_sp.at[idx_s.at[j]], sem, add=True)
        pltpu.async_copy(ones_v, din_sp.at[idx_d.at[j]], sem2, add=True)

        @pl.when(j >= 1)
        def _():
            pltpu.make_async_copy(ones_v, dout_sp.at[pl.ds(0, CW)], sem).wait()
            pltpu.make_async_copy(ones_v, din_sp.at[pl.ds(0, CW)], sem2).wait()
        return carry

    lax.fori_loop(0, K, body, 0)
    pltpu.make_async_copy(ones_v, dout_sp.at[pl.ds(0, CW)], sem).wait()
    pltpu.make_async_copy(ones_v, din_sp.at[pl.ds(0, CW)], sem2).wait()
    plsc.subcore_barrier()
    pltpu.sync_copy(dout_sp.at[pl.ds(s * ZD, ZD)], out_hbm.at[c, 0, pl.ds(s * ZD, ZD)])
    pltpu.sync_copy(din_sp.at[pl.ds(s * ZD, ZD)], out_hbm.at[c, 1, pl.ds(s * ZD, ZD)])


_deg_kernel = functools.partial(
    pl.kernel,
    out_type=jax.ShapeDtypeStruct((NC, 2, NPAD), _f32),
    mesh=_mesh,
    scratch_types=[
        pltpu.VMEM((K, CW), jnp.int32),
        pltpu.VMEM((K, CW), jnp.int32),
        pltpu.VMEM((CW,), _f32),
        pltpu.VMEM_SHARED((NPAD,), _f32),
        pltpu.VMEM_SHARED((NPAD,), _f32),
        pltpu.SemaphoreType.DMA,
        pltpu.SemaphoreType.DMA,
    ],
)(_deg_body)


def _agg_body(y_hbm, src4d, dst4d, zrows, out_hbm, idx_s, idx_d, rows_v,
              agg_sp, semg0, semg1, semg2, semg3, semc0, semc1, semc2, semc3):
    c = lax.axis_index("c")
    s = lax.axis_index("s")
    w = s * NC + c
    pltpu.sync_copy(zrows.at[pl.ds(s * ZR, ZR)], agg_sp.at[pl.ds(s * ZR, ZR)])
    plsc.subcore_barrier()

    semg = (semg0, semg1, semg2, semg3)
    semc = (semc0, semc1, semc2, semc3)

    def _drain(b):
        pltpu.make_async_copy(zrows.at[pl.ds(0, CWA)], rows_v.at[b],
                              semc[b]).wait()

    def _step(j, p):
        q3 = (p + 3) % 4

        @pl.when(j + 3 < WK)
        def _():
            @pl.when(j >= 1)
            def _():
                _drain(q3)
            pltpu.async_copy(y_hbm.at[idx_s.at[j + 3]], rows_v.at[q3],
                             semg[q3])

        pltpu.make_async_copy(y_hbm.at[idx_s.at[j]], rows_v.at[p],
                              semg[p]).wait()
        pltpu.async_copy(rows_v.at[p], agg_sp.at[idx_d.at[j]], semc[p],
                         add=True)

    def body(j, carry):
        for p in range(4):
            @pl.when(j % 4 == p)
            def _(p=p):
                _step(j, p)
        return carry

    def window(t, carry):
        pltpu.sync_copy(src4d.at[w, t], idx_s)
        pltpu.sync_copy(dst4d.at[w, t], idx_d)
        for jj in range(3):
            pltpu.async_copy(y_hbm.at[idx_s.at[jj]], rows_v.at[jj], semg[jj])
        lax.fori_loop(0, WK, body, 0)
        for r in range(WK - 4, WK):
            _drain(r % 4)
        return carry

    lax.fori_loop(0, WT, window, 0)
    plsc.subcore_barrier()
    pltpu.sync_copy(agg_sp.at[pl.ds(s * ZR, ZR)], out_hbm.at[c, pl.ds(s * ZR, ZR)])


_agg_kernel = functools.partial(
    pl.kernel,
    out_type=jax.ShapeDtypeStruct((NC, NPA, D), _f32),
    mesh=_mesh,
    scratch_types=[
        pltpu.VMEM((WK, CWA), jnp.int32),
        pltpu.VMEM((WK, CWA), jnp.int32),
        pltpu.VMEM((4, CWA, D), _f32),
        pltpu.VMEM_SHARED((NPA, D), _f32),
        pltpu.SemaphoreType.DMA,
        pltpu.SemaphoreType.DMA,
        pltpu.SemaphoreType.DMA,
        pltpu.SemaphoreType.DMA,
        pltpu.SemaphoreType.DMA,
        pltpu.SemaphoreType.DMA,
        pltpu.SemaphoreType.DMA,
        pltpu.SemaphoreType.DMA,
    ],
)(_agg_body)


def _link_body(p_hbm, q_hbm, srcd, dstd, out_hbm, idx_s, idx_d, rows_v,
               semg0, semg1, semg2, sema0, sema1, sema2, semw0, semw1, semw2):
    c = lax.axis_index("c")
    s = lax.axis_index("s")
    w = s * NC + c
    pltpu.sync_copy(srcd.at[w], idx_s)
    pltpu.sync_copy(dstd.at[w], idx_d)

    semg = (semg0, semg1, semg2)
    sema = (sema0, sema1, sema2)
    semw = (semw0, semw1, semw2)

    pltpu.async_copy(p_hbm.at[idx_s.at[0]], rows_v.at[0], semg0)

    def _step(j, p):
        q = (p + 1) % 3

        @pl.when(j + 1 < K5)
        def _():
            @pl.when(j >= 2)
            def _():
                pltpu.make_async_copy(
                    rows_v.at[q],
                    out_hbm.at[pl.ds(w * EPT5 + (j - 2) * CW, CW)],
                    semw[q]).wait()
            pltpu.async_copy(p_hbm.at[idx_s.at[j + 1]], rows_v.at[q], semg[q])

        pltpu.make_async_copy(p_hbm.at[idx_s.at[j]], rows_v.at[p], semg[p]).wait()
        pltpu.async_copy(q_hbm.at[idx_d.at[j]], rows_v.at[p], sema[p],
                         add=True).wait()
        pltpu.async_copy(rows_v.at[p], out_hbm.at[pl.ds(w * EPT5 + j * CW, CW)],
                         semw[p])

    def body(j, carry):
        for p in range(3):
            @pl.when(j % 3 == p)
            def _(p=p):
                _step(j, p)
        return carry

    lax.fori_loop(0, K5, body, 0)
    pltpu.make_async_copy(rows_v.at[(K5 - 2) % 3],
                          out_hbm.at[pl.ds(w * EPT5 + (K5 - 2) * CW, CW)],
                          semw[(K5 - 2) % 3]).wait()
    pltpu.make_async_copy(rows_v.at[(K5 - 1) % 3],
                          out_hbm.at[pl.ds(w * EPT5 + (K5 - 1) * CW, CW)],
                          semw[(K5 - 1) % 3]).wait()


_link_kernel = functools.partial(
    pl.kernel,
    out_type=jax.ShapeDtypeStruct((ESL, D), _f32),
    mesh=_mesh,
    scratch_types=[
        pltpu.VMEM((K5, CW), jnp.int32),
        pltpu.VMEM((K5, CW), jnp.int32),
        pltpu.VMEM((3, CW, D), _f32),
        pltpu.SemaphoreType.DMA,
        pltpu.SemaphoreType.DMA,
        pltpu.SemaphoreType.DMA,
        pltpu.SemaphoreType.DMA,
        pltpu.SemaphoreType.DMA,
        pltpu.SemaphoreType.DMA,
        pltpu.SemaphoreType.DMA,
        pltpu.SemaphoreType.DMA,
        pltpu.SemaphoreType.DMA,
    ],
)(_link_body)


_RN = 1000
_RE = 6400


def _tc0_body(h_ref, we_ref, be_ref, x_ref):
    x_ref[...] = jnp.dot(h_ref[...], we_ref[...],
                         preferred_element_type=_f32) + be_ref[...]


def _tc1b_body(x_ref, degt_ref, y1_ref, ns_ref, nd_ref):
    dblk = degt_ref[...]
    ns = lax.rsqrt(jnp.maximum(dblk[:, 0:1] + dblk[:, 2:3], 1.0))
    nd = lax.rsqrt(jnp.maximum(dblk[:, 1:2] + dblk[:, 3:4], 1.0))
    y1_ref[...] = x_ref[...] * ns
    ns_ref[...] = ns
    nd_ref[...] = nd


def _tc2_body(aggp_ref, x_ref, ns_ref, nd_ref, w_ref, gb_ref, x2_ref, y2_ref):
    agg = aggp_ref[0] + aggp_ref[1]
    out = jnp.dot(agg * nd_ref[...], w_ref[...], preferred_element_type=_f32)
    r = jnp.maximum(out * gb_ref[0:1, :] + gb_ref[1:2, :], 0.0)
    x2 = x_ref[...] + r
    x2_ref[...] = x2
    y2_ref[...] = x2 * ns_ref[...]


def _tc3a_body(aggp_ref, x_ref, nd_ref, w_ref, gb_ref,
               m0a_ref, m0b_ref, mb0_ref,
               p_ref, qm_ref, x3_ref):
    agg = aggp_ref[0] + aggp_ref[1]
    out = jnp.dot(agg * nd_ref[...], w_ref[...], preferred_element_type=_f32)
    r = jnp.maximum(out * gb_ref[0:1, :] + gb_ref[1:2, :], 0.0)
    x3 = x_ref[...] + r
    x3_ref[...] = x3
    p_ref[...] = jnp.dot(x3, m0a_ref[...], preferred_element_type=_f32) + mb0_ref[...]
    qm_ref[...] = jnp.dot(x3, m0b_ref[...], preferred_element_type=_f32)


def _tc3b_body(x3_ref, q0_ref, qb0_ref, q1_ref, qb1_ref, q2_ref, qb2_ref,
               nscore_ref):
    x3 = x3_ref[...]
    yq = jnp.maximum(jnp.dot(x3, q0_ref[...], preferred_element_type=_f32) + qb0_ref[...], 0.0)
    yq = jnp.maximum(jnp.dot(yq, q1_ref[...], preferred_element_type=_f32) + qb1_ref[...], 0.0)
    nscore_ref[...] = jax.nn.sigmoid(
        jnp.dot(yq, q2_ref[...], preferred_element_type=_f32) + qb2_ref[...])


def _tc4_body(pre_ref, m1_ref, mb1_ref, m2_ref, mb2_ref, out_ref):
    ym = jnp.maximum(pre_ref[...], 0.0).astype(jnp.bfloat16)
    h1 = jnp.maximum(
        jnp.dot(ym, m1_ref[...], preferred_element_type=_f32) + mb1_ref[...], 0.0)
    z = jnp.dot(h1, m2_ref[...], preferred_element_type=_f32) + mb2_ref[...]
    out_ref[...] = jax.nn.sigmoid(z.reshape(1, _RE // D, D))


def _row_spec(r, cols):
    return pl.BlockSpec((r, cols), lambda i: (i, 0))


def _full_spec(shape):
    nd = len(shape)
    return pl.BlockSpec(shape, lambda i: (0,) * nd)


def kernel(h, e, W_emb, b_emb, W1, b1, g1, be1, W2, b2, g2, be2,
           M0, mb0, M1, mb1, M2, mb2, Q0, qb0, Q1, qb1, Q2, qb2, edge_index):
    del e
    src = edge_index[0]
    dst = edge_index[1]
    src2d = src.reshape(NW, K, CW)
    dst2d = dst.reshape(NW, K, CW)
    npad_e = EPTA - EPT
    pad_i = jnp.arange(NW * npad_e, dtype=jnp.int32).reshape(NW, npad_e)
    pad_src = (pad_i * 41) % N
    pad_dst = N + (pad_i % (NPA - N))
    src4d = jnp.concatenate([src.reshape(NW, EPT), pad_src], axis=1)
    src4d = src4d.reshape(NW, WT, WK, CWA)
    dst4d = jnp.concatenate([dst.reshape(NW, EPT), pad_dst], axis=1)
    dst4d = dst4d.reshape(NW, WT, WK, CWA)
    zeros1d = jnp.zeros((NPAD,), _f32)
    zrows = jnp.zeros((NPA, D), _f32)
    bn_scale = 1.0 / jnp.sqrt(jnp.float32(1.0 + 1e-5))
    g1e = g1 * bn_scale
    gb1 = jnp.stack([g1e, b1 * g1e + be1])
    g2e = g2 * bn_scale
    gb2 = jnp.stack([g2e, b2 * g2e + be2])

    deg_parts = _deg_kernel(src2d, dst2d, zeros1d)
    deg_t = deg_parts.reshape(4, NPAD).T[:N]

    grid_n = N // _RN
    x = pl.pallas_call(
        _tc0_body,
        grid=(grid_n,),
        in_specs=[_row_spec(_RN, D), _full_spec((D, D)), _full_spec((1, D))],
        out_specs=_row_spec(_RN, D),
        out_shape=jax.ShapeDtypeStruct((N, D), _f32),
    )(h, W_emb, b_emb.reshape(1, D))

    y1, ns, nd = pl.pallas_call(
        _tc1b_body,
        grid=(grid_n,),
        in_specs=[_row_spec(_RN, D), _row_spec(_RN, 4)],
        out_specs=[
            _row_spec(_RN, D), _row_spec(_RN, 1), _row_spec(_RN, 1),
        ],
        out_shape=[
            jax.ShapeDtypeStruct((N, D), _f32),
            jax.ShapeDtypeStruct((N, 1), _f32),
            jax.ShapeDtypeStruct((N, 1), _f32),
        ],
    )(x, deg_t)

    agg1 = _agg_kernel(y1, src4d, dst4d, zrows)
    x2, y2 = pl.pallas_call(
        _tc2_body,
        grid=(grid_n,),
        in_specs=[
            pl.BlockSpec((NC, _RN, D), lambda i: (0, i, 0)),
            _row_spec(_RN, D), _row_spec(_RN, 1), _row_spec(_RN, 1),
            _full_spec((D, D)), _full_spec((2, D)),
        ],
        out_specs=[_row_spec(_RN, D), _row_spec(_RN, D)],
        out_shape=[
            jax.ShapeDtypeStruct((N, D), _f32),
            jax.ShapeDtypeStruct((N, D), _f32),
        ],
    )(agg1, x, ns, nd, W1, gb1)

    agg2 = _agg_kernel(y2, src4d, dst4d, zrows)
    P, Q, x3 = pl.pallas_call(
        _tc3a_body,
        grid=(grid_n,),
        in_specs=[
            pl.BlockSpec((NC, _RN, D), lambda i: (0, i, 0)),
            _row_spec(_RN, D), _row_spec(_RN, 1),
            _full_spec((D, D)), _full_spec((2, D)),
            _full_spec((D, D)), _full_spec((D, D)), _full_spec((1, D)),
        ],
        out_specs=[_row_spec(_RN, D), _row_spec(_RN, D), _row_spec(_RN, D)],
        out_shape=[
            jax.ShapeDtypeStruct((N, D), _f32),
            jax.ShapeDtypeStruct((N, D), _f32),
            jax.ShapeDtypeStruct((N, D), _f32),
        ],
    )(agg2, x2, nd, W2, gb2, M0[:D], M0[D:], mb0.reshape(1, D))

    src5d = src.reshape(NSPL, NW, K5, CW)
    dst5d = dst.reshape(NSPL, NW, K5, CW)
    m1_bf = M1.astype(jnp.bfloat16)
    zs = []
    for sl in range(NSPL):
        pre_sl = _link_kernel(P, Q, src5d[sl], dst5d[sl])
        zs.append(pl.pallas_call(
            _tc4_body,
            grid=(ESL // _RE,),
            in_specs=[
                _row_spec(_RE, D),
                _full_spec((D, D // 2)), _full_spec((1, D // 2)),
                _full_spec((D // 2, 1)), _full_spec((1, 1)),
            ],
            out_specs=pl.BlockSpec((1, _RE // D, D), lambda i: (i, 0, 0)),
            out_shape=jax.ShapeDtypeStruct((ESL // _RE, _RE // D, D), _f32),
        )(pre_sl, m1_bf, mb1.reshape(1, -1), M2, mb2.reshape(1, 1)))

    nscore = pl.pallas_call(
        _tc3b_body,
        grid=(grid_n,),
        in_specs=[
            _row_spec(_RN, D),
            _full_spec((D, D // 2)), _full_spec((1, D // 2)),
            _full_spec((D // 2, D // 4)), _full_spec((1, D // 4)),
            _full_spec((D // 4, 1)), _full_spec((1, 1)),
        ],
        out_specs=_row_spec(_RN, 1),
        out_shape=jax.ShapeDtypeStruct((N, 1), _f32),
    )(x3, Q0, qb0.reshape(1, -1), Q1, qb1.reshape(1, -1), Q2, qb2.reshape(1, -1))

    link = jnp.concatenate(zs, axis=0).reshape(E, 1)

    return (link, nscore)
